# baseline (speedup 1.0000x reference)
import functools

import jax
import jax.numpy as jnp
from jax import lax
from jax.experimental import pallas as pl
from jax.experimental.pallas import tpu as pltpu
from jax.experimental.pallas import tpu_sc as plsc

L = 16
ROWS = 64
N = 32768
NVEC = N // L
K = 64
NB = K // L
NW = 32
ROWS_PER_W = ROWS // NW
BLK = 16
NBLK = NVEC // BLK


def _iota():
    return lax.broadcasted_iota(jnp.int32, (L,), 0)


def _take(x, idx):
    return x.at[idx].get(mode="promise_in_bounds")


def _lex_gt(av, ai, bv, bi):
    return (av > bv) | ((av == bv) & (ai < bi))


def _cmpx(v, i, perm, keep_max):
    pv = _take(v, perm)
    pi = _take(i, perm)
    keep_self = _lex_gt(v, i, pv, pi) == keep_max
    return jnp.where(keep_self, v, pv), jnp.where(keep_self, i, pi)


def _bitonic_clean16(v, i, consts):
    for perm, keep_max in consts["clean"]:
        v, i = _cmpx(v, i, perm, keep_max)
    return v, i


def _merge16(av, ai, bv, bi, consts, clean_lo=True):
    rbv = jnp.flip(bv)
    rbi = jnp.flip(bi)
    gt = _lex_gt(av, ai, rbv, rbi)
    hv = jnp.where(gt, av, rbv)
    hi = jnp.where(gt, ai, rbi)
    hv, hi = _bitonic_clean16(hv, hi, consts)
    if not clean_lo:
        return hv, hi, None, None
    lv = jnp.where(gt, rbv, av)
    li = jnp.where(gt, rbi, ai)
    lv, li = _bitonic_clean16(lv, li, consts)
    return hv, hi, lv, li


def _make_consts():
    iota = _iota()
    clean_c = []
    for d in (8, 4, 2, 1):
        clean_c.append((iota ^ d, (iota & d) == 0))
    return {"clean": tuple(clean_c), "iota": iota,
            "last": jnp.full((L,), L - 1, jnp.int32),
            "shift": jnp.maximum(iota - 1, 0)}


def _suffix_merge(tvs, tis, cv, ci, start, consts):
    tvs, tis = list(tvs), list(tis)
    for b in range(start, NB):
        last = b == NB - 1
        hv, hi, cv, ci = _merge16(tvs[b], tis[b], cv, ci, consts,
                                  clean_lo=not last)
        tvs[b], tis[b] = hv, hi
    return tuple(tvs), tuple(tis)


def _insert_one(args2, v, idx, consts):
    tvs = list(args2[0:4])
    tis = list(args2[4:8])
    m = v > args2[8]
    lane = plsc.all_reduce_ffs(m)
    sv = _take(v, lane)
    si = _take(idx, lane)
    iota = consts["iota"]
    for b in range(NB):
        a, ai = tvs[b], tis[b]
        gt = _lex_gt(a, ai, sv, si)
        cnt = plsc.all_reduce_population_count(gt)
        tprev = _take(a, consts["shift"])
        tiprev = _take(ai, consts["shift"])
        a15 = _take(a, consts["last"])
        ai15 = _take(ai, consts["last"])
        ins = iota == cnt
        tvs[b] = jnp.where(gt, a, jnp.where(ins, sv, tprev))
        tis[b] = jnp.where(gt, ai, jnp.where(ins, si, tiprev))
        full = cnt == L
        sv = jnp.where(full, sv, a15)
        si = jnp.where(full, si, ai15)
    nthr = _take(tvs[3], consts["last"])
    return tuple(tvs) + tuple(tis) + (nthr,)


def _tree_max(vs):
    vs = list(vs)
    while len(vs) > 1:
        vs = [jnp.maximum(vs[i], vs[i + 1]) for i in range(0, len(vs), 2)]
    return vs[0]


def _make_rescan(row_v, consts):
    def rescan(j, state):
        def vec_body(k, carry2):
            thr2 = carry2[8]
            off = pl.multiple_of(j * (BLK * L) + k * L, L)
            v = row_v[pl.ds(off, L)]
            nh = plsc.all_reduce_population_count(v > thr2)
            idx = (j * BLK + k) * L + consts["iota"]

            def chunk_merge(args2):
                tvs = args2[0:4]
                tis = args2[4:8]
                cv, ci = plsc.sort_key_val(v, idx, descending=True)
                cmax = cv[0]
                g0 = tvs[0][L - 1]
                g1 = tvs[1][L - 1]
                g2 = tvs[2][L - 1]

                def from_b(b):
                    def f(_):
                        ntv, nti = _suffix_merge(tvs, tis, cv, ci, b, consts)
                        nthr = _take(ntv[3], consts["last"])
                        return ntv + nti + (nthr,)
                    return f

                return lax.cond(
                    cmax < g2, from_b(3),
                    lambda u: lax.cond(
                        cmax < g1, from_b(2),
                        lambda u2: lax.cond(
                            cmax < g0, from_b(1), from_b(0), u2),
                        u),
                    0)

            def merge(args2):
                return lax.cond(
                    nh[0] == 1,
                    lambda a: _insert_one(a, v, idx, consts),
                    chunk_merge, args2)

            return lax.cond(nh[0] > 0, merge, lambda a: a, carry2)

        return lax.fori_loop(0, BLK, vec_body, state)

    return rescan


def _scan_two_rows(row_a, row_b, consts):
    neg_inf = jnp.full((L,), -jnp.inf, jnp.float32)
    big_idx = jnp.full((L,), jnp.iinfo(jnp.int32).max, jnp.int32)
    init9 = (neg_inf, neg_inf, neg_inf, neg_inf,
             big_idx, big_idx, big_idx, big_idx,
             neg_inf)
    rescan_a = _make_rescan(row_a, consts)
    rescan_b = _make_rescan(row_b, consts)

    def blk_body(j, carry):
        thra = carry[8]
        thrb = carry[17]
        base = pl.multiple_of(j * (BLK * L), BLK * L)
        mxa = _tree_max([row_a[pl.ds(base + k * L, L)] for k in range(BLK)])
        mxb = _tree_max([row_b[pl.ds(base + k * L, L)] for k in range(BLK)])
        pa = plsc.all_reduce_population_count(mxa > thra)
        pb = plsc.all_reduce_population_count(mxb > thrb)

        def trig(c):
            ca = lax.cond(pa[0] > 0, lambda s: rescan_a(j, s),
                          lambda s: s, c[0:9])
            cb = lax.cond(pb[0] > 0, lambda s: rescan_b(j, s),
                          lambda s: s, c[9:18])
            return ca + cb

        return lax.cond(pa[0] + pb[0] > 0, trig, lambda c: c, carry)

    out = lax.fori_loop(0, NBLK, blk_body, init9 + init9)
    return out[0:9], out[9:18]


def _topk_body(a_hbm, vals_hbm, idxs_hbm, row_v0, row_v1, outv_v, outi_v,
               sem0, sem1):
    consts = _make_consts()
    wid = lax.axis_index("s") * 2 + lax.axis_index("c")
    row0 = wid * ROWS_PER_W
    row1 = row0 + 1

    cp0 = pltpu.async_copy(a_hbm.at[row0], row_v0, sem0)
    cp1 = pltpu.async_copy(a_hbm.at[row1], row_v1, sem1)
    cp0.wait()
    cp1.wait()

    outa, outb = _scan_two_rows(row_v0, row_v1, consts)
    for row, out in ((row0, outa), (row1, outb)):
        for b in range(NB):
            outv_v[pl.ds(b * L, L)] = out[b]
            outi_v[pl.ds(b * L, L)] = out[4 + b]
        pltpu.sync_copy(outv_v, vals_hbm.at[row])
        pltpu.sync_copy(outi_v, idxs_hbm.at[row])


@functools.partial(
    pl.kernel,
    mesh=plsc.VectorSubcoreMesh(core_axis_name="c", subcore_axis_name="s"),
    out_type=[
        jax.ShapeDtypeStruct((ROWS, K), jnp.float32),
        jax.ShapeDtypeStruct((ROWS, K), jnp.int32),
    ],
    scratch_types=[
        pltpu.VMEM((N,), jnp.float32),
        pltpu.VMEM((N,), jnp.float32),
        pltpu.VMEM((K,), jnp.float32),
        pltpu.VMEM((K,), jnp.int32),
        pltpu.SemaphoreType.DMA,
        pltpu.SemaphoreType.DMA,
    ],
    compiler_params=pltpu.CompilerParams(needs_layout_passes=False),
)
def _topk_sc(a_hbm, vals_hbm, idxs_hbm, row_v0, row_v1, outv_v, outi_v,
             sem0, sem1):
    _topk_body(a_hbm, vals_hbm, idxs_hbm, row_v0, row_v1, outv_v, outi_v,
               sem0, sem1)


def kernel(a_tensor, value_tensor, indice_tensor):
    values, indices = _topk_sc(a_tensor)
    return values, indices


# speedup vs baseline: 1.1641x; 1.1641x over previous
import functools

import jax
import jax.numpy as jnp
from jax import lax
from jax.experimental import pallas as pl
from jax.experimental.pallas import tpu as pltpu
from jax.experimental.pallas import tpu_sc as plsc

L = 16
ROWS = 64
N = 32768
NVEC = N // L
K = 64
NB = K // L
NW = 32
ROWS_PER_W = ROWS // NW
BLK = 8
NBLK = NVEC // BLK


def _iota():
    return lax.broadcasted_iota(jnp.int32, (L,), 0)


def _take(x, idx):
    return x.at[idx].get(mode="promise_in_bounds")


def _lex_gt(av, ai, bv, bi):
    return (av > bv) | ((av == bv) & (ai < bi))


def _cmpx(v, i, perm, keep_max):
    pv = _take(v, perm)
    pi = _take(i, perm)
    keep_self = _lex_gt(v, i, pv, pi) == keep_max
    return jnp.where(keep_self, v, pv), jnp.where(keep_self, i, pi)


def _bitonic_clean16(v, i, consts):
    for perm, keep_max in consts["clean"]:
        v, i = _cmpx(v, i, perm, keep_max)
    return v, i


def _merge16(av, ai, bv, bi, consts, clean_lo=True):
    rbv = jnp.flip(bv)
    rbi = jnp.flip(bi)
    gt = _lex_gt(av, ai, rbv, rbi)
    hv = jnp.where(gt, av, rbv)
    hi = jnp.where(gt, ai, rbi)
    hv, hi = _bitonic_clean16(hv, hi, consts)
    if not clean_lo:
        return hv, hi, None, None
    lv = jnp.where(gt, rbv, av)
    li = jnp.where(gt, rbi, ai)
    lv, li = _bitonic_clean16(lv, li, consts)
    return hv, hi, lv, li


def _make_consts():
    iota = _iota()
    clean_c = []
    for d in (8, 4, 2, 1):
        clean_c.append((iota ^ d, (iota & d) == 0))
    return {"clean": tuple(clean_c), "iota": iota,
            "last": jnp.full((L,), L - 1, jnp.int32),
            "shift": jnp.maximum(iota - 1, 0)}


def _suffix_merge(tvs, tis, cv, ci, start, consts):
    tvs, tis = list(tvs), list(tis)
    for b in range(start, NB):
        last = b == NB - 1
        hv, hi, cv, ci = _merge16(tvs[b], tis[b], cv, ci, consts,
                                  clean_lo=not last)
        tvs[b], tis[b] = hv, hi
    return tuple(tvs), tuple(tis)


def _insert_one(args2, v, idx, consts):
    tvs = args2[0:4]
    tis = args2[4:8]
    m = v > args2[8]
    lane = plsc.all_reduce_ffs(m)
    sv = _take(v, lane)
    si = _take(idx, lane)
    iota = consts["iota"]
    cnts = [plsc.all_reduce_population_count(_lex_gt(tvs[b], tis[b], sv, si))
            for b in range(NB)]
    pos = cnts[0] + cnts[1] + cnts[2] + cnts[3]
    ntv, nti = [], []
    for b in range(NB):
        a, ai = tvs[b], tis[b]
        rel = pos - L * b
        if b > 0:
            a15p = _take(tvs[b - 1], consts["last"])
            ai15p = _take(tis[b - 1], consts["last"])
        else:
            a15p, ai15p = sv, si
        tprev = jnp.where(iota == 0, a15p, _take(a, consts["shift"]))
        tiprev = jnp.where(iota == 0, ai15p, _take(ai, consts["shift"]))
        in_b = (rel >= 0) & (rel < L)
        noshift = rel >= L
        keep = noshift | (in_b & (iota < rel))
        ins = in_b & (iota == rel)
        ntv.append(jnp.where(keep, a, jnp.where(ins, sv, tprev)))
        nti.append(jnp.where(keep, ai, jnp.where(ins, si, tiprev)))
    nthr = _take(ntv[3], consts["last"])
    return tuple(ntv) + tuple(nti) + (nthr,)


def _tree_max(vs):
    vs = list(vs)
    while len(vs) > 1:
        vs = [jnp.maximum(vs[i], vs[i + 1]) for i in range(0, len(vs), 2)]
    return vs[0]


def _make_rescan(row_v, consts):
    def rescan(j, state):
        def vec_body(k, carry2):
            thr2 = carry2[8]
            off = pl.multiple_of(j * (BLK * L) + k * L, L)
            v = row_v[pl.ds(off, L)]
            nh = plsc.all_reduce_population_count(v > thr2)
            idx = (j * BLK + k) * L + consts["iota"]

            def chunk_merge(args2):
                tvs = args2[0:4]
                tis = args2[4:8]
                cv, ci = plsc.sort_key_val(v, idx, descending=True)
                cmax = cv[0]
                g0 = tvs[0][L - 1]
                g1 = tvs[1][L - 1]
                g2 = tvs[2][L - 1]

                def from_b(b):
                    def f(_):
                        ntv, nti = _suffix_merge(tvs, tis, cv, ci, b, consts)
                        nthr = _take(ntv[3], consts["last"])
                        return ntv + nti + (nthr,)
                    return f

                return lax.cond(
                    cmax < g2, from_b(3),
                    lambda u: lax.cond(
                        cmax < g1, from_b(2),
                        lambda u2: lax.cond(
                            cmax < g0, from_b(1), from_b(0), u2),
                        u),
                    0)

            def merge(args2):
                return lax.cond(
                    nh[0] == 1,
                    lambda a: _insert_one(a, v, idx, consts),
                    chunk_merge, args2)

            return lax.cond(nh[0] > 0, merge, lambda a: a, carry2)

        return lax.fori_loop(0, BLK, vec_body, state)

    return rescan


def _scan_two_rows(row_a, row_b, consts):
    neg_inf = jnp.full((L,), -jnp.inf, jnp.float32)
    big_idx = jnp.full((L,), jnp.iinfo(jnp.int32).max, jnp.int32)
    init9 = (neg_inf, neg_inf, neg_inf, neg_inf,
             big_idx, big_idx, big_idx, big_idx,
             neg_inf)
    rescan_a = _make_rescan(row_a, consts)
    rescan_b = _make_rescan(row_b, consts)

    def blk_body(j, carry):
        thra = carry[8]
        thrb = carry[17]
        base = pl.multiple_of(j * (BLK * L), BLK * L)
        mxa = _tree_max([row_a[pl.ds(base + k * L, L)] for k in range(BLK)])
        mxb = _tree_max([row_b[pl.ds(base + k * L, L)] for k in range(BLK)])
        pa = plsc.all_reduce_population_count(mxa > thra)
        pb = plsc.all_reduce_population_count(mxb > thrb)

        def trig(c):
            ca = lax.cond(pa[0] > 0, lambda s: rescan_a(j, s),
                          lambda s: s, c[0:9])
            cb = lax.cond(pb[0] > 0, lambda s: rescan_b(j, s),
                          lambda s: s, c[9:18])
            return ca + cb

        return lax.cond(pa[0] + pb[0] > 0, trig, lambda c: c, carry)

    out = lax.fori_loop(0, NBLK, blk_body, init9 + init9)
    return out[0:9], out[9:18]


def _topk_body(a_hbm, vals_hbm, idxs_hbm, row_v0, row_v1, outv_v, outi_v,
               sem0, sem1):
    consts = _make_consts()
    wid = lax.axis_index("s") * 2 + lax.axis_index("c")
    row0 = wid * ROWS_PER_W
    row1 = row0 + 1

    cp0 = pltpu.async_copy(a_hbm.at[row0], row_v0, sem0)
    cp1 = pltpu.async_copy(a_hbm.at[row1], row_v1, sem1)
    cp0.wait()
    cp1.wait()

    outa, outb = _scan_two_rows(row_v0, row_v1, consts)
    for row, out in ((row0, outa), (row1, outb)):
        for b in range(NB):
            outv_v[pl.ds(b * L, L)] = out[b]
            outi_v[pl.ds(b * L, L)] = out[4 + b]
        pltpu.sync_copy(outv_v, vals_hbm.at[row])
        pltpu.sync_copy(outi_v, idxs_hbm.at[row])


@functools.partial(
    pl.kernel,
    mesh=plsc.VectorSubcoreMesh(core_axis_name="c", subcore_axis_name="s"),
    out_type=[
        jax.ShapeDtypeStruct((ROWS, K), jnp.float32),
        jax.ShapeDtypeStruct((ROWS, K), jnp.int32),
    ],
    scratch_types=[
        pltpu.VMEM((N,), jnp.float32),
        pltpu.VMEM((N,), jnp.float32),
        pltpu.VMEM((K,), jnp.float32),
        pltpu.VMEM((K,), jnp.int32),
        pltpu.SemaphoreType.DMA,
        pltpu.SemaphoreType.DMA,
    ],
    compiler_params=pltpu.CompilerParams(needs_layout_passes=False),
)
def _topk_sc(a_hbm, vals_hbm, idxs_hbm, row_v0, row_v1, outv_v, outi_v,
             sem0, sem1):
    _topk_body(a_hbm, vals_hbm, idxs_hbm, row_v0, row_v1, outv_v, outi_v,
               sem0, sem1)


def kernel(a_tensor, value_tensor, indice_tensor):
    values, indices = _topk_sc(a_tensor)
    return values, indices


# speedup vs baseline: 1.1723x; 1.0071x over previous
import functools

import jax
import jax.numpy as jnp
from jax import lax
from jax.experimental import pallas as pl
from jax.experimental.pallas import tpu as pltpu
from jax.experimental.pallas import tpu_sc as plsc

L = 16
ROWS = 64
N = 32768
NVEC = N // L
K = 64
NB = K // L
NW = 32
ROWS_PER_W = ROWS // NW
BLK = 8
NBLK = NVEC // BLK


def _iota():
    return lax.broadcasted_iota(jnp.int32, (L,), 0)


def _take(x, idx):
    return x.at[idx].get(mode="promise_in_bounds")


def _lex_gt(av, ai, bv, bi):
    return (av > bv) | ((av == bv) & (ai < bi))


def _cmpx(v, i, perm, keep_max):
    pv = _take(v, perm)
    pi = _take(i, perm)
    keep_self = _lex_gt(v, i, pv, pi) == keep_max
    return jnp.where(keep_self, v, pv), jnp.where(keep_self, i, pi)


def _bitonic_clean16(v, i, consts):
    for perm, keep_max in consts["clean"]:
        v, i = _cmpx(v, i, perm, keep_max)
    return v, i


def _merge16(av, ai, bv, bi, consts, clean_lo=True):
    rbv = jnp.flip(bv)
    rbi = jnp.flip(bi)
    gt = _lex_gt(av, ai, rbv, rbi)
    hv = jnp.where(gt, av, rbv)
    hi = jnp.where(gt, ai, rbi)
    hv, hi = _bitonic_clean16(hv, hi, consts)
    if not clean_lo:
        return hv, hi, None, None
    lv = jnp.where(gt, rbv, av)
    li = jnp.where(gt, rbi, ai)
    lv, li = _bitonic_clean16(lv, li, consts)
    return hv, hi, lv, li


def _make_consts():
    iota = _iota()
    clean_c = []
    for d in (8, 4, 2, 1):
        clean_c.append((iota ^ d, (iota & d) == 0))
    return {"clean": tuple(clean_c), "iota": iota,
            "last": jnp.full((L,), L - 1, jnp.int32),
            "shift": jnp.maximum(iota - 1, 0)}


def _suffix_merge(tvs, tis, cv, ci, start, consts):
    tvs, tis = list(tvs), list(tis)
    for b in range(start, NB):
        last = b == NB - 1
        hv, hi, cv, ci = _merge16(tvs[b], tis[b], cv, ci, consts,
                                  clean_lo=not last)
        tvs[b], tis[b] = hv, hi
    return tuple(tvs), tuple(tis)


def _insert_one(args2, v, idx, consts):
    tvs = args2[0:4]
    tis = args2[4:8]
    m = v > args2[8]
    lane = plsc.all_reduce_ffs(m)
    sv = _take(v, lane)
    si = _take(idx, lane)
    iota = consts["iota"]
    cnts = [plsc.all_reduce_population_count(_lex_gt(tvs[b], tis[b], sv, si))
            for b in range(NB)]
    pos = cnts[0] + cnts[1] + cnts[2] + cnts[3]
    ntv, nti = [], []
    for b in range(NB):
        a, ai = tvs[b], tis[b]
        rel = pos - L * b
        if b > 0:
            a15p = _take(tvs[b - 1], consts["last"])
            ai15p = _take(tis[b - 1], consts["last"])
        else:
            a15p, ai15p = sv, si
        tprev = jnp.where(iota == 0, a15p, _take(a, consts["shift"]))
        tiprev = jnp.where(iota == 0, ai15p, _take(ai, consts["shift"]))
        in_b = (rel >= 0) & (rel < L)
        noshift = rel >= L
        keep = noshift | (in_b & (iota < rel))
        ins = in_b & (iota == rel)
        ntv.append(jnp.where(keep, a, jnp.where(ins, sv, tprev)))
        nti.append(jnp.where(keep, ai, jnp.where(ins, si, tiprev)))
    nthr = _take(ntv[3], consts["last"])
    return tuple(ntv) + tuple(nti) + (nthr,)


def _tree_max(vs):
    vs = list(vs)
    while len(vs) > 1:
        vs = [jnp.maximum(vs[i], vs[i + 1]) for i in range(0, len(vs), 2)]
    return vs[0]


def _make_rescan(row_v, consts):
    def rescan(j, state):
        def vec_body(k, carry2):
            thr2 = carry2[8]
            off = pl.multiple_of(j * (BLK * L) + k * L, L)
            v = row_v[pl.ds(off, L)]
            nh = plsc.all_reduce_population_count(v > thr2)
            idx = (j * BLK + k) * L + consts["iota"]

            def chunk_merge(args2):
                tvs = args2[0:4]
                tis = args2[4:8]
                cv, ci = plsc.sort_key_val(v, idx, descending=True)
                cmax = cv[0]
                g0 = tvs[0][L - 1]
                g1 = tvs[1][L - 1]
                g2 = tvs[2][L - 1]

                def from_b(b):
                    def f(_):
                        ntv, nti = _suffix_merge(tvs, tis, cv, ci, b, consts)
                        nthr = _take(ntv[3], consts["last"])
                        return ntv + nti + (nthr,)
                    return f

                return lax.cond(
                    cmax < g2, from_b(3),
                    lambda u: lax.cond(
                        cmax < g1, from_b(2),
                        lambda u2: lax.cond(
                            cmax < g0, from_b(1), from_b(0), u2),
                        u),
                    0)

            def merge(args2):
                return lax.cond(
                    nh[0] == 1,
                    lambda a: _insert_one(a, v, idx, consts),
                    chunk_merge, args2)

            return lax.cond(nh[0] > 0, merge, lambda a: a, carry2)

        def pair_body(p, carry2):
            thr2 = carry2[8]
            off = pl.multiple_of(j * (BLK * L) + p * (2 * L), 2 * L)
            v0 = row_v[pl.ds(off, L)]
            v1 = row_v[pl.ds(off + L, L)]
            mp = jnp.maximum(v0, v1)
            nhp = plsc.all_reduce_population_count(mp > thr2)

            def handle(a):
                return lax.fori_loop(2 * p, 2 * p + 2, vec_body, a)

            return lax.cond(nhp[0] > 0, handle, lambda a: a, carry2)

        return lax.fori_loop(0, BLK // 2, pair_body, state)

    return rescan


def _scan_two_rows(row_a, row_b, consts):
    neg_inf = jnp.full((L,), -jnp.inf, jnp.float32)
    big_idx = jnp.full((L,), jnp.iinfo(jnp.int32).max, jnp.int32)
    init9 = (neg_inf, neg_inf, neg_inf, neg_inf,
             big_idx, big_idx, big_idx, big_idx,
             neg_inf)
    rescan_a = _make_rescan(row_a, consts)
    rescan_b = _make_rescan(row_b, consts)

    def blk_body(j, carry):
        thra = carry[8]
        thrb = carry[17]
        base = pl.multiple_of(j * (BLK * L), BLK * L)
        mxa = _tree_max([row_a[pl.ds(base + k * L, L)] for k in range(BLK)])
        mxb = _tree_max([row_b[pl.ds(base + k * L, L)] for k in range(BLK)])
        pa = plsc.all_reduce_population_count(mxa > thra)
        pb = plsc.all_reduce_population_count(mxb > thrb)

        def trig(c):
            ca = lax.cond(pa[0] > 0, lambda s: rescan_a(j, s),
                          lambda s: s, c[0:9])
            cb = lax.cond(pb[0] > 0, lambda s: rescan_b(j, s),
                          lambda s: s, c[9:18])
            return ca + cb

        return lax.cond(pa[0] + pb[0] > 0, trig, lambda c: c, carry)

    out = lax.fori_loop(0, NBLK, blk_body, init9 + init9)
    return out[0:9], out[9:18]


def _topk_body(a_hbm, vals_hbm, idxs_hbm, row_v0, row_v1, outv_v, outi_v,
               sem0, sem1):
    consts = _make_consts()
    wid = lax.axis_index("s") * 2 + lax.axis_index("c")
    row0 = wid * ROWS_PER_W
    row1 = row0 + 1

    cp0 = pltpu.async_copy(a_hbm.at[row0], row_v0, sem0)
    cp1 = pltpu.async_copy(a_hbm.at[row1], row_v1, sem1)
    cp0.wait()
    cp1.wait()

    outa, outb = _scan_two_rows(row_v0, row_v1, consts)
    for row, out in ((row0, outa), (row1, outb)):
        for b in range(NB):
            outv_v[pl.ds(b * L, L)] = out[b]
            outi_v[pl.ds(b * L, L)] = out[4 + b]
        pltpu.sync_copy(outv_v, vals_hbm.at[row])
        pltpu.sync_copy(outi_v, idxs_hbm.at[row])


@functools.partial(
    pl.kernel,
    mesh=plsc.VectorSubcoreMesh(core_axis_name="c", subcore_axis_name="s"),
    out_type=[
        jax.ShapeDtypeStruct((ROWS, K), jnp.float32),
        jax.ShapeDtypeStruct((ROWS, K), jnp.int32),
    ],
    scratch_types=[
        pltpu.VMEM((N,), jnp.float32),
        pltpu.VMEM((N,), jnp.float32),
        pltpu.VMEM((K,), jnp.float32),
        pltpu.VMEM((K,), jnp.int32),
        pltpu.SemaphoreType.DMA,
        pltpu.SemaphoreType.DMA,
    ],
    compiler_params=pltpu.CompilerParams(needs_layout_passes=False),
)
def _topk_sc(a_hbm, vals_hbm, idxs_hbm, row_v0, row_v1, outv_v, outi_v,
             sem0, sem1):
    _topk_body(a_hbm, vals_hbm, idxs_hbm, row_v0, row_v1, outv_v, outi_v,
               sem0, sem1)


def kernel(a_tensor, value_tensor, indice_tensor):
    values, indices = _topk_sc(a_tensor)
    return values, indices
